# baseline (speedup 1.0000x reference)
import functools

import jax
import jax.numpy as jnp
from jax import lax
from jax.experimental import pallas as pl
from jax.experimental.pallas import tpu as pltpu
from jax.experimental.pallas import tpu_sc as plsc

N = 100000
D = 128
HID = 16
H = 4
B = 64
HP = 8
BLK = 2000
NB = N // BLK

NW = 32
UNIT = 400
UPB = BLK // UNIT
NU = N // UNIT
SREPS = (NU + NW - 1) // NW
UG = 5
GPU = UNIT // 16
HU = H * UNIT

_NEG = float("-inf")


def _score_body(x_ref, seg_ref, w1_ref, b1_ref, w2_ref, sc_ref, sc2_ref,
                smax_ref):
    i = pl.program_id(0)
    x = x_ref[...]
    hid = jnp.maximum(
        jnp.dot(x, w1_ref[...], preferred_element_type=jnp.float32)
        + b1_ref[...], 0.0)
    st = lax.dot_general(w2_ref[...], hid, (((1,), (1,)), ((), ())),
                         preferred_element_type=jnp.float32)
    sc_ref[0] = st
    for j in range(UPB):
        sc2_ref[0, j] = st[:H, j * UNIT:(j + 1) * UNIT]
    seg = seg_ref[0]
    bids = lax.broadcasted_iota(jnp.int32, (B, BLK), 0)
    mask = bids == seg
    ci = lax.broadcasted_iota(jnp.int32, (B, HP), 1)
    bm = jnp.full((B, HP), _NEG)
    for h in range(H):
        mh = jnp.max(jnp.where(mask, st[h:h + 1, :], _NEG),
                     axis=1, keepdims=True)
        bm = jnp.where(ci == h, mh, bm)

    @pl.when(i == 0)
    def _():
        smax_ref[...] = bm

    @pl.when(i > 0)
    def _():
        smax_ref[...] = jnp.maximum(smax_ref[...], bm)


def _scores_and_segmax(x, seg3, w1cat, b1row, w2t):
    return pl.pallas_call(
        _score_body,
        grid=(NB,),
        in_specs=[
            pl.BlockSpec((BLK, D), lambda i: (i, 0)),
            pl.BlockSpec((1, 1, BLK), lambda i: (i, 0, 0)),
            pl.BlockSpec((D, H * HID), lambda i: (0, 0)),
            pl.BlockSpec((1, H * HID), lambda i: (0, 0)),
            pl.BlockSpec((HP, H * HID), lambda i: (0, 0)),
        ],
        out_specs=[
            pl.BlockSpec((1, HP, BLK), lambda i: (i, 0, 0)),
            pl.BlockSpec((1, UPB, H, UNIT), lambda i: (i, 0, 0, 0)),
            pl.BlockSpec((B, HP), lambda i: (0, 0)),
        ],
        out_shape=[
            jax.ShapeDtypeStruct((NB, HP, BLK), jnp.float32),
            jax.ShapeDtypeStruct((NB, UPB, H, UNIT), jnp.float32),
            jax.ShapeDtypeStruct((B, HP), jnp.float32),
        ],
    )(x, seg3, w1cat, b1row, w2t)


def _denom_body(sc_hbm, seg_hbm, smax_hbm, out_hbm, *scratch):
    seg_bufs = scratch[0:SREPS]
    sc_bufs = scratch[SREPS:2 * SREPS]
    smax_v = scratch[2 * SREPS]
    acc_v = scratch[2 * SREPS + 1]
    sems = scratch[2 * SREPS + 2:]
    c = lax.axis_index("c")
    s = lax.axis_index("s")
    wid = s * 2 + c
    pltpu.sync_copy(smax_hbm, smax_v)
    z = jnp.zeros((16,), jnp.float32)
    for j in range((HP * B) // 16):
        acc_v[pl.ds(j * 16, 16)] = z

    for r in range(SREPS):
        u = wid + r * NW

        @pl.when(u < NU)
        def _(u=u, r=r):
            pltpu.async_copy(seg_hbm.at[pl.ds(u * UNIT, UNIT)],
                             seg_bufs[r], sems[r])
            pltpu.async_copy(sc_hbm.at[u // UPB, u % UPB],
                             sc_bufs[r], sems[r])

    for r in range(SREPS):
        u = wid + r * NW

        @pl.when(u < NU)
        def _(u=u, r=r):
            pltpu.make_async_copy(seg_hbm.at[pl.ds(0, UNIT)],
                                  seg_bufs[r], sems[r]).wait()
            pltpu.make_async_copy(sc_hbm.at[0, 0],
                                  sc_bufs[r], sems[r]).wait()
            seg_v = seg_bufs[r]
            sc_v = sc_bufs[r]

            def body_g(g5, carry):
                for k in range(UG):
                    base = (g5 * UG + k) * 16
                    sv = seg_v[pl.ds(base, 16)]
                    svp = sv * HP
                    for h in range(H):
                        s16 = sc_v[h, pl.ds(base, 16)]
                        iv = svp + h
                        m16 = plsc.load_gather(smax_v, [iv])
                        e = jnp.exp(s16 - m16)
                        plsc.addupdate_scatter(acc_v, [iv], e)
                return carry

            lax.fori_loop(0, GPU // UG, body_g, 0)

    pltpu.sync_copy(acc_v, out_hbm.at[wid])


@functools.partial(jax.jit, static_argnums=())
def _denom_partials(scores_t, seg, smax):
    mesh = plsc.VectorSubcoreMesh(core_axis_name="c", subcore_axis_name="s")
    k = functools.partial(
        pl.kernel,
        mesh=mesh,
        compiler_params=pltpu.CompilerParams(needs_layout_passes=False),
        out_type=jax.ShapeDtypeStruct((NW, HP * B), jnp.float32),
        scratch_types=(
            [pltpu.VMEM((UNIT,), jnp.int32)] * SREPS
            + [pltpu.VMEM((H, UNIT), jnp.float32)] * SREPS
            + [pltpu.VMEM((B * HP,), jnp.float32),
               pltpu.VMEM((HP * B,), jnp.float32)]
            + [pltpu.SemaphoreType.DMA] * SREPS
        ),
    )(_denom_body)
    return k(scores_t, seg, smax)


def _pool_body(x_ref, seg_ref, sc_ref, smax_ref, parts_ref, out_ref):
    i = pl.program_id(0)
    d = jnp.sum(parts_ref[...], axis=0)
    sm = smax_ref[...]
    sm = jnp.where(jnp.isfinite(sm), sm, 0.0)
    z = jnp.where(d > 0, sm + jnp.log(d), 1e30)
    seg = seg_ref[0]
    oh = (lax.broadcasted_iota(jnp.int32, (B, BLK), 0) == seg
          ).astype(jnp.float32)
    zg = lax.dot_general(z, oh, (((0,), (0,)), ((), ())),
                         preferred_element_type=jnp.float32,
                         precision=lax.Precision.HIGHEST)
    e = jnp.exp(sc_ref[0] - zg)
    a = jnp.sum(e, axis=0, keepdims=True) * (1.0 / H)
    w = oh * a
    part = lax.dot_general(w, x_ref[...], (((1,), (0,)), ((), ())),
                           preferred_element_type=jnp.float32)

    @pl.when(i == 0)
    def _():
        out_ref[...] = part

    @pl.when(i > 0)
    def _():
        out_ref[...] += part


def _pool(x, seg3, scores_t, smax_t, parts):
    return pl.pallas_call(
        _pool_body,
        grid=(NB,),
        in_specs=[
            pl.BlockSpec((BLK, D), lambda i: (i, 0)),
            pl.BlockSpec((1, 1, BLK), lambda i: (i, 0, 0)),
            pl.BlockSpec((1, HP, BLK), lambda i: (i, 0, 0)),
            pl.BlockSpec((B, HP), lambda i: (0, 0)),
            pl.BlockSpec((NW, B, HP), lambda i: (0, 0, 0)),
        ],
        out_specs=pl.BlockSpec((B, D), lambda i: (0, 0)),
        out_shape=jax.ShapeDtypeStruct((B, D), jnp.float32),
    )(x, seg3, scores_t, smax_t, parts)


def kernel(node_features, segment_ids, W1, b1, W2, b2):
    x = node_features.astype(jnp.float32)
    seg = segment_ids.astype(jnp.int32)
    seg3 = seg.reshape(NB, 1, BLK)

    w1cat = jnp.transpose(W1, (1, 0, 2)).reshape(D, H * HID)
    b1row = b1.reshape(1, H * HID)
    w2r = W2[:, :, 0]
    w2t = jnp.zeros((HP, H * HID), jnp.float32)
    for h in range(H):
        w2t = w2t.at[h, h * HID:(h + 1) * HID].set(w2r[h])

    scores_t, scores_u, smax = _scores_and_segmax(x, seg3, w1cat, b1row, w2t)
    parts = _denom_partials(scores_u, seg, smax.reshape(B * HP))
    out = _pool(x, seg3, scores_t, smax, parts.reshape(NW, B, HP))
    return out


# speedup vs baseline: 1.0562x; 1.0562x over previous
import functools

import jax
import jax.numpy as jnp
from jax import lax
from jax.experimental import pallas as pl
from jax.experimental.pallas import tpu as pltpu
from jax.experimental.pallas import tpu_sc as plsc

N = 100000
D = 128
HID = 16
H = 4
B = 64
HP = 8
BLK = 2000
NB = N // BLK

NW = 32
UNIT = 400
UPB = BLK // UNIT
NU = N // UNIT
SREPS = (NU + NW - 1) // NW
UG = 5
GPU = UNIT // 16
HU = H * UNIT

_NEG = float("-inf")


def _score_body(x_ref, seg_ref, w1_ref, b1_ref, w2_ref, sc_ref, sc2_ref,
                smax_ref):
    i = pl.program_id(0)
    x = x_ref[...]
    hid = jnp.maximum(
        jnp.dot(x, w1_ref[...], preferred_element_type=jnp.float32)
        + b1_ref[...], 0.0)
    st = lax.dot_general(w2_ref[...], hid, (((1,), (1,)), ((), ())),
                         preferred_element_type=jnp.float32)
    sc_ref[0] = st
    for j in range(UPB):
        sc2_ref[0, j] = st[:H, j * UNIT:(j + 1) * UNIT]
    seg = seg_ref[0]
    bids = lax.broadcasted_iota(jnp.int32, (B, BLK), 0)
    mask = bids == seg
    ci = lax.broadcasted_iota(jnp.int32, (B, HP), 1)
    bm = jnp.full((B, HP), _NEG)
    for h in range(H):
        mh = jnp.max(jnp.where(mask, st[h:h + 1, :], _NEG),
                     axis=1, keepdims=True)
        bm = jnp.where(ci == h, mh, bm)

    @pl.when(i == 0)
    def _():
        smax_ref[...] = bm

    @pl.when(i > 0)
    def _():
        smax_ref[...] = jnp.maximum(smax_ref[...], bm)


def _scores_and_segmax(x, seg3, w1cat, b1row, w2t):
    return pl.pallas_call(
        _score_body,
        grid=(NB,),
        in_specs=[
            pl.BlockSpec((BLK, D), lambda i: (i, 0)),
            pl.BlockSpec((1, 1, BLK), lambda i: (i, 0, 0)),
            pl.BlockSpec((D, H * HID), lambda i: (0, 0)),
            pl.BlockSpec((1, H * HID), lambda i: (0, 0)),
            pl.BlockSpec((HP, H * HID), lambda i: (0, 0)),
        ],
        out_specs=[
            pl.BlockSpec((1, HP, BLK), lambda i: (i, 0, 0)),
            pl.BlockSpec((1, UPB, H, UNIT), lambda i: (i, 0, 0, 0)),
            pl.BlockSpec((B, HP), lambda i: (0, 0)),
        ],
        out_shape=[
            jax.ShapeDtypeStruct((NB, HP, BLK), jnp.float32),
            jax.ShapeDtypeStruct((NB, UPB, H, UNIT), jnp.float32),
            jax.ShapeDtypeStruct((B, HP), jnp.float32),
        ],
    )(x, seg3, w1cat, b1row, w2t)


def _denom_body(sc_hbm, seg_hbm, smax_hbm, out_hbm, *scratch):
    seg_bufs = scratch[0:SREPS]
    sc_bufs = scratch[SREPS:2 * SREPS]
    smax_v = scratch[2 * SREPS]
    acc_v = scratch[2 * SREPS + 1]
    sems = scratch[2 * SREPS + 2:]
    c = lax.axis_index("c")
    s = lax.axis_index("s")
    wid = s * 2 + c
    pltpu.sync_copy(smax_hbm, smax_v)
    z = jnp.zeros((16,), jnp.float32)
    for j in range((HP * B) // 16):
        acc_v[pl.ds(j * 16, 16)] = z

    for r in range(SREPS):
        u = wid + r * NW

        @pl.when(u < NU)
        def _(u=u, r=r):
            pltpu.async_copy(seg_hbm.at[pl.ds(u * UNIT, UNIT)],
                             seg_bufs[r], sems[r])
            pltpu.async_copy(sc_hbm.at[u // UPB, u % UPB],
                             sc_bufs[r], sems[r])

    for r in range(SREPS):
        u = wid + r * NW

        @pl.when(u < NU)
        def _(u=u, r=r):
            pltpu.make_async_copy(seg_hbm.at[pl.ds(0, UNIT)],
                                  seg_bufs[r], sems[r]).wait()
            pltpu.make_async_copy(sc_hbm.at[0, 0],
                                  sc_bufs[r], sems[r]).wait()
            seg_v = seg_bufs[r]
            sc_v = sc_bufs[r]

            def body_g(g5, carry):
                for k in range(UG):
                    base = (g5 * UG + k) * 16
                    sv = seg_v[pl.ds(base, 16)]
                    svp = sv * HP
                    for h in range(H):
                        s16 = sc_v[h, pl.ds(base, 16)]
                        iv = svp + h
                        m16 = plsc.load_gather(smax_v, [iv])
                        e = jnp.exp(s16 - m16)
                        plsc.addupdate_scatter(acc_v, [iv], e)
                return carry

            lax.fori_loop(0, GPU // UG, body_g, 0)

    pltpu.sync_copy(acc_v, out_hbm.at[wid])


@functools.partial(jax.jit, static_argnums=())
def _denom_partials(scores_t, seg, smax):
    mesh = plsc.VectorSubcoreMesh(core_axis_name="c", subcore_axis_name="s")
    k = functools.partial(
        pl.kernel,
        mesh=mesh,
        compiler_params=pltpu.CompilerParams(needs_layout_passes=False),
        out_type=jax.ShapeDtypeStruct((NW, HP * B), jnp.float32),
        scratch_types=(
            [pltpu.VMEM((UNIT,), jnp.int32)] * SREPS
            + [pltpu.VMEM((H, UNIT), jnp.float32)] * SREPS
            + [pltpu.VMEM((B * HP,), jnp.float32),
               pltpu.VMEM((HP * B,), jnp.float32)]
            + [pltpu.SemaphoreType.DMA] * SREPS
        ),
    )(_denom_body)
    return k(scores_t, seg, smax)


def _uacc_body(x_ref, seg_ref, sc_ref, smax_ref, u_ref):
    i = pl.program_id(0)
    sm = smax_ref[...]
    sm = jnp.where(jnp.isfinite(sm), sm, 0.0)
    seg = seg_ref[0]
    oh = (lax.broadcasted_iota(jnp.int32, (B, BLK), 0) == seg
          ).astype(jnp.float32)
    smg = lax.dot_general(sm, oh, (((0,), (0,)), ((), ())),
                          preferred_element_type=jnp.float32,
                          precision=lax.Precision.HIGHEST)
    e = jnp.exp(sc_ref[0] - smg)
    for h in range(H):
        w = oh * e[h:h + 1, :]
        part = lax.dot_general(w, x_ref[...], (((1,), (0,)), ((), ())),
                               preferred_element_type=jnp.float32)

        @pl.when(i == 0)
        def _(h=h, part=part):
            u_ref[h] = part

        @pl.when(i > 0)
        def _(h=h, part=part):
            u_ref[h] += part


def _uacc(x, seg3, scores_t, smax):
    return pl.pallas_call(
        _uacc_body,
        grid=(NB,),
        in_specs=[
            pl.BlockSpec((BLK, D), lambda i: (i, 0)),
            pl.BlockSpec((1, 1, BLK), lambda i: (i, 0, 0)),
            pl.BlockSpec((1, HP, BLK), lambda i: (i, 0, 0)),
            pl.BlockSpec((B, HP), lambda i: (0, 0)),
        ],
        out_specs=pl.BlockSpec((H, B, D), lambda i: (0, 0, 0)),
        out_shape=jax.ShapeDtypeStruct((H, B, D), jnp.float32),
    )(x, seg3, scores_t, smax)


def _finish_body(u_ref, parts_ref, out_ref):
    d = jnp.sum(parts_ref[...], axis=0)
    dinv = jnp.where(d > 0, 1.0 / d, 0.0)
    acc = u_ref[0] * dinv[:, 0:1]
    for h in range(1, H):
        acc = acc + u_ref[h] * dinv[:, h:h + 1]
    out_ref[...] = acc * (1.0 / H)


def _finish(u, parts):
    return pl.pallas_call(
        _finish_body,
        in_specs=[
            pl.BlockSpec((H, B, D), lambda: (0, 0, 0)),
            pl.BlockSpec((NW, B, HP), lambda: (0, 0, 0)),
        ],
        out_specs=pl.BlockSpec((B, D), lambda: (0, 0)),
        out_shape=jax.ShapeDtypeStruct((B, D), jnp.float32),
    )(u, parts)


def kernel(node_features, segment_ids, W1, b1, W2, b2):
    x = node_features.astype(jnp.float32)
    seg = segment_ids.astype(jnp.int32)
    seg3 = seg.reshape(NB, 1, BLK)

    w1cat = jnp.transpose(W1, (1, 0, 2)).reshape(D, H * HID)
    b1row = b1.reshape(1, H * HID)
    w2r = W2[:, :, 0]
    w2t = jnp.zeros((HP, H * HID), jnp.float32)
    for h in range(H):
        w2t = w2t.at[h, h * HID:(h + 1) * HID].set(w2r[h])

    scores_t, scores_u, smax = _scores_and_segmax(x, seg3, w1cat, b1row, w2t)
    parts = _denom_partials(scores_u, seg, smax.reshape(B * HP))
    u = _uacc(x, seg3, scores_t, smax)
    out = _finish(u, parts.reshape(NW, B, HP))
    return out
